# baseline (speedup 1.0000x reference)
import jax
import jax.numpy as jnp
from jax import lax
from jax.experimental import pallas as pl
from jax.experimental.pallas import tpu as pltpu
from jax.experimental.pallas import tpu_sc as plsc

_NC = 2
_NS = 16
_NW = _NC * _NS

_BATCH = 16384
_FIELDS = 26
_DIM = 32
_ROWS = _BATCH * _FIELDS
_ROWS_PER_W = _ROWS // _NW
_CHUNK_ROWS = 1664
_N_CHUNKS = _ROWS_PER_W // _CHUNK_ROWS


def _gather_body(tbl_hbm, idx_hbm, out_hbm, idx_v, buf0, buf1, gsem0, gsem1,
                 ssem0, ssem1):
    wid = lax.axis_index("s") * _NC + lax.axis_index("c")
    row0 = wid * _ROWS_PER_W
    pltpu.sync_copy(idx_hbm.at[pl.ds(row0, _ROWS_PER_W)], idx_v)

    bufs = (buf0, buf1)
    gsems = (gsem0, gsem1)
    ssems = (ssem0, ssem1)

    def fire_gather(i):
        p = i % 2
        return pltpu.async_copy(
            tbl_hbm.at[idx_v.at[pl.ds(i * _CHUNK_ROWS, _CHUNK_ROWS)]],
            bufs[p], gsems[p])

    def fire_store(i):
        p = i % 2
        return pltpu.async_copy(
            bufs[p], out_hbm.at[pl.ds(row0 + i * _CHUNK_ROWS, _CHUNK_ROWS)],
            ssems[p])

    gathers = [None] * _N_CHUNKS
    stores = [None] * _N_CHUNKS
    gathers[0] = fire_gather(0)
    for i in range(_N_CHUNKS):
        gathers[i].wait()
        if i >= 1:
            stores[i - 1].wait()
        if i + 1 < _N_CHUNKS:
            gathers[i + 1] = fire_gather(i + 1)
        stores[i] = fire_store(i)
    stores[_N_CHUNKS - 1].wait()


_gather = pl.kernel(
    _gather_body,
    out_type=jax.ShapeDtypeStruct((_ROWS, _DIM), jnp.float32),
    mesh=plsc.VectorSubcoreMesh(
        core_axis_name="c", subcore_axis_name="s",
        num_cores=_NC, num_subcores=_NS,
    ),
    scratch_types=[
        pltpu.VMEM((_ROWS_PER_W,), jnp.int32),
        pltpu.VMEM((_CHUNK_ROWS, _DIM), jnp.float32),
        pltpu.VMEM((_CHUNK_ROWS, _DIM), jnp.float32),
        pltpu.SemaphoreType.DMA,
        pltpu.SemaphoreType.DMA,
        pltpu.SemaphoreType.DMA,
        pltpu.SemaphoreType.DMA,
    ],
    compiler_params=pltpu.CompilerParams(use_tc_tiling_on_sc=False),
)


def kernel(input_tensor, weight):
    idx_flat = input_tensor.astype(jnp.int32).reshape(_ROWS)
    out = _gather(weight, idx_flat)
    return out.reshape(_BATCH, _FIELDS, _DIM)
